# Initial kernel scaffold; baseline (speedup 1.0000x reference)
import jax
import jax.numpy as jnp
from jax.experimental import pallas as pl


def kernel(x, edge_index, batch, We, be, Wg1, bg1, Wg2, bg2, W2, b2, gamma1, beta1, gamma2, beta2, Wc, bc):
    raise NotImplementedError("write your pallas kernel here")


# speedup vs baseline: 15.9377x; 15.9377x over previous
import functools

import jax
import jax.numpy as jnp
from jax import lax
from jax.experimental import pallas as pl
from jax.experimental.pallas import tpu as pltpu
from jax.experimental.pallas import tpu_sc as plsc

N = 10000
E = 320000
D = 128
H = 64
NUM_EXPERTS = 3
NUM_CLASSES = 3
NUM_GRAPHS = 128

NC = 2
NS = 16
NW = NC * NS

CH = 128
CHUNKS = -(-E // (NW * CH))
EPT = CHUNKS * CH
EP = EPT * NW

NP = 10240
RPS = NP // NS
NPT = NP // NW
SEG = 136

_mesh = plsc.VectorSubcoreMesh(
    core_axis_name="c", subcore_axis_name="s", num_cores=NC, num_subcores=NS)
_sc_params = pltpu.CompilerParams(
    use_tc_tiling_on_sc=False, needs_layout_passes=False)


def _fill_rows(ref, nrows, ncols, val):
    vv = jnp.full((16,), val, jnp.float32)

    def body(i, carry):
        for cc in range(ncols // 16):
            ref[i, pl.ds(cc * 16, 16)] = vv
        return carry

    lax.fori_loop(0, nrows, body, 0)


@functools.partial(
    pl.kernel,
    out_type=jax.ShapeDtypeStruct((NC, NP, 16), jnp.float32),
    mesh=_mesh,
    scratch_types=[
        pltpu.VMEM((CH,), jnp.int32),
        pltpu.VMEM((CH, 16), jnp.float32),
        pltpu.VMEM((CH, 16), jnp.float32),
        pltpu.VMEM_SHARED((NP, 16), jnp.float32),
    ],
    compiler_params=_sc_params,
)
def _deg_kernel(dst_hbm, out_hbm, dst_v, ones_v, zbuf, acc):
    c = lax.axis_index("c")
    s = lax.axis_index("s")
    wid = c * NS + s

    iot = lax.iota(jnp.int32, 16)
    e0 = jnp.where(iot == 0, 1.0, 0.0).astype(jnp.float32)

    def initb(i, carry):
        ones_v[i, pl.ds(0, 16)] = e0
        zbuf[i, pl.ds(0, 16)] = jnp.zeros((16,), jnp.float32)
        return carry

    lax.fori_loop(0, CH, initb, 0)
    for j in range(RPS // CH):
        pltpu.sync_copy(zbuf, acc.at[pl.ds(s * RPS + j * CH, CH)])
    plsc.subcore_barrier()

    def body(g, carry):
        off = wid * EPT + g * CH
        pltpu.sync_copy(dst_hbm.at[pl.ds(off, CH)], dst_v)
        pltpu.sync_copy(ones_v, acc.at[dst_v], add=True)
        return carry

    lax.fori_loop(0, CHUNKS, body, 0)
    plsc.subcore_barrier()

    for j in range(RPS // CH):
        r0 = s * RPS + j * CH
        pltpu.sync_copy(acc.at[pl.ds(r0, CH)], zbuf)
        pltpu.sync_copy(zbuf, out_hbm.at[c].at[pl.ds(r0, CH)])


def _make_edge_scatter(W):
    @functools.partial(
        pl.kernel,
        out_type=jax.ShapeDtypeStruct((NC, NP, W), jnp.float32),
        mesh=_mesh,
        scratch_types=[
            pltpu.VMEM((CH,), jnp.int32),
            pltpu.VMEM((CH,), jnp.int32),
            pltpu.VMEM((CH, W), jnp.float32),
            pltpu.VMEM_SHARED((NP, W), jnp.float32),
            pltpu.SemaphoreType.DMA,
        ],
        compiler_params=_sc_params,
    )
    def scat(xs_hbm, src_hbm, dst_hbm, out_hbm, src_v, dst_v, rows, acc, sem):
        c = lax.axis_index("c")
        s = lax.axis_index("s")
        wid = c * NS + s

        _fill_rows(rows, CH, W, 0.0)
        for j in range(RPS // CH):
            pltpu.sync_copy(rows, acc.at[pl.ds(s * RPS + j * CH, CH)])
        plsc.subcore_barrier()

        def body(g, carry):
            off = wid * EPT + g * CH
            pltpu.sync_copy(src_hbm.at[pl.ds(off, CH)], src_v)
            pltpu.async_copy(xs_hbm.at[src_v], rows, sem).wait()
            pltpu.sync_copy(dst_hbm.at[pl.ds(off, CH)], dst_v)
            pltpu.sync_copy(rows, acc.at[dst_v], add=True)
            return carry

        lax.fori_loop(0, CHUNKS, body, 0)
        plsc.subcore_barrier()

        for j in range(RPS // CH):
            r0 = s * RPS + j * CH
            pltpu.sync_copy(acc.at[pl.ds(r0, CH)], rows)
            pltpu.sync_copy(rows, out_hbm.at[c].at[pl.ds(r0, CH)])

    return scat


_edge_scatter_128 = _make_edge_scatter(128)
_edge_scatter_64 = _make_edge_scatter(64)


@functools.partial(
    pl.kernel,
    out_type=(
        jax.ShapeDtypeStruct((NW, SEG, H), jnp.float32),
        jax.ShapeDtypeStruct((NW, SEG, H), jnp.float32),
        jax.ShapeDtypeStruct((NW, SEG, 16), jnp.float32),
    ),
    mesh=_mesh,
    scratch_types=[
        pltpu.VMEM((64, H), jnp.float32),
        pltpu.VMEM((NPT,), jnp.int32),
        pltpu.VMEM((SEG, H), jnp.float32),
        pltpu.VMEM((SEG, H), jnp.float32),
        pltpu.VMEM((SEG, 16), jnp.float32),
    ],
    compiler_params=_sc_params,
)
def _pool_kernel(h2_hbm, batch_hbm, osum, omax, ocnt, hbuf, bbuf, lsum, lmax, lcnt):
    c = lax.axis_index("c")
    s = lax.axis_index("s")
    wid = c * NS + s
    base = wid * NPT

    _fill_rows(lsum, SEG, H, 0.0)
    _fill_rows(lmax, SEG, H, -jnp.inf)
    _fill_rows(lcnt, SEG, 16, 0.0)
    pltpu.sync_copy(batch_hbm.at[pl.ds(base, NPT)], bbuf)

    iot = lax.iota(jnp.int32, 16)
    e0 = jnp.where(iot == 0, 1.0, 0.0).astype(jnp.float32)

    for j in range(NPT // 64):
        pltpu.sync_copy(h2_hbm.at[pl.ds(base + j * 64, 64)], hbuf)

        def body(i, carry):
            bsp = plsc.load_gather(bbuf, [jnp.full((16,), j * 64 + i, jnp.int32)])
            for cc in range(H // 16):
                cols = cc * 16 + iot
                hrow = hbuf[i, pl.ds(cc * 16, 16)]
                cur = plsc.load_gather(lsum, [bsp, cols])
                plsc.store_scatter(lsum, [bsp, cols], cur + hrow)
                curm = plsc.load_gather(lmax, [bsp, cols])
                plsc.store_scatter(lmax, [bsp, cols], jnp.maximum(curm, hrow))
            cc0 = plsc.load_gather(lcnt, [bsp, iot])
            plsc.store_scatter(lcnt, [bsp, iot], cc0 + e0)
            return carry

        lax.fori_loop(0, 64, body, 0)

    pltpu.sync_copy(lsum, osum.at[wid])
    pltpu.sync_copy(lmax, omax.at[wid])
    pltpu.sync_copy(lcnt, ocnt.at[wid])


def _tc1_body(x_ref, wg1_ref, bg1_ref, wg2_ref, bg2_ref, degp_ref,
              xs_ref, dinv_ref, gates_ref):
    x = x_ref[...]
    t = jnp.maximum(x @ wg1_ref[...] + bg1_ref[...][None, :], 0.0)
    logits = t @ wg2_ref[...] + bg2_ref[...][None, :]
    m = jnp.max(logits, axis=1, keepdims=True)
    ez = jnp.exp(logits - m)
    gates_ref[...] = ez / jnp.sum(ez, axis=1, keepdims=True)
    deg = jnp.sum(degp_ref[...][:, :N, :], axis=(0, 2)) + 1.0
    dinv = lax.rsqrt(deg)
    dinv_ref[...] = dinv[:, None]
    xs_ref[...] = x * dinv[:, None]


def _tc2_body(accp_ref, xs_ref, dinv_ref, gates_ref, wef_ref, be_ref,
              g1_ref, b1_ref, hs_ref):
    agg = (accp_ref[0, :N, :] + accp_ref[1, :N, :] + xs_ref[...]) * dinv_ref[...]
    eo = agg @ wef_ref[...]
    gates = gates_ref[...]
    h = (gates[:, 0:1] * eo[:, 0:H] + gates[:, 1:2] * eo[:, H:2 * H]
         + gates[:, 2:3] * eo[:, 2 * H:3 * H]) + gates @ be_ref[...]
    mean = jnp.mean(h, axis=0, keepdims=True)
    var = jnp.mean((h - mean) ** 2, axis=0, keepdims=True)
    hn = (h - mean) * lax.rsqrt(var + 1e-5) * g1_ref[...][None, :] + b1_ref[...][None, :]
    hs_ref[...] = jnp.maximum(hn, 0.0) * dinv_ref[...]


def _tc3_body(accp_ref, hs_ref, dinv_ref, w2_ref, b2_ref, g2_ref, be2_ref,
              h2p_ref):
    agg = (accp_ref[0, :N, :] + accp_ref[1, :N, :] + hs_ref[...]) * dinv_ref[...]
    h2 = agg @ w2_ref[...] + b2_ref[...][None, :]
    mean = jnp.mean(h2, axis=0, keepdims=True)
    var = jnp.mean((h2 - mean) ** 2, axis=0, keepdims=True)
    hn = (h2 - mean) * lax.rsqrt(var + 1e-5) * g2_ref[...][None, :] + be2_ref[...][None, :]
    h2p_ref[...] = jnp.zeros((NP, H), jnp.float32)
    h2p_ref[:N, :] = jnp.maximum(hn, 0.0)


def _tc4_body(osum_ref, omax_ref, ocnt_ref, wc_ref, bc_ref, out_ref):
    ssum = jnp.sum(osum_ref[...], axis=0)[:NUM_GRAPHS]
    smax = jnp.max(omax_ref[...], axis=0)[:NUM_GRAPHS]
    cnt = jnp.sum(ocnt_ref[...], axis=(0, 2))[:NUM_GRAPHS]
    smean = ssum / jnp.maximum(cnt, 1.0)[:, None]
    smax = jnp.where((cnt > 0.0)[:, None], smax, 0.0)
    g = jnp.concatenate([smean, smax, ssum], axis=1)
    out_ref[...] = g @ wc_ref[...] + bc_ref[...][None, :]


def kernel(x, edge_index, batch, We, be, Wg1, bg1, Wg2, bg2, W2, b2,
           gamma1, beta1, gamma2, beta2, Wc, bc):
    src = edge_index[0]
    dst = edge_index[1]
    padn = EP - E
    srcp = jnp.concatenate([src, jnp.zeros((padn,), jnp.int32)])
    dstp = jnp.concatenate([dst, jnp.full((padn,), N, jnp.int32)])
    batchp = jnp.concatenate([batch, jnp.full((NP - N,), NUM_GRAPHS, jnp.int32)])
    Wef = jnp.transpose(We, (1, 0, 2)).reshape(D, NUM_EXPERTS * H)

    degp = _deg_kernel(dstp)

    xs, dinv, gates = pl.pallas_call(
        _tc1_body,
        out_shape=(
            jax.ShapeDtypeStruct((N, D), jnp.float32),
            jax.ShapeDtypeStruct((N, 1), jnp.float32),
            jax.ShapeDtypeStruct((N, NUM_EXPERTS), jnp.float32),
        ),
    )(x, Wg1, bg1, Wg2, bg2, degp)

    accp1 = _edge_scatter_128(xs, srcp, dstp)

    hs = pl.pallas_call(
        _tc2_body,
        out_shape=jax.ShapeDtypeStruct((N, H), jnp.float32),
    )(accp1, xs, dinv, gates, Wef, be, gamma1, beta1)

    accp2 = _edge_scatter_64(hs, srcp, dstp)

    h2p = pl.pallas_call(
        _tc3_body,
        out_shape=jax.ShapeDtypeStruct((NP, H), jnp.float32),
    )(accp2, hs, dinv, W2, b2, gamma2, beta2)

    osum, omax, ocnt = _pool_kernel(h2p, batchp)

    out = pl.pallas_call(
        _tc4_body,
        out_shape=jax.ShapeDtypeStruct((NUM_GRAPHS, NUM_CLASSES), jnp.float32),
    )(osum, omax, ocnt, Wc, bc)
    return out
